# baseline (speedup 1.0000x reference)
import functools

import jax
import jax.numpy as jnp
from jax import lax
from jax.experimental import pallas as pl
from jax.experimental.pallas import tpu as pltpu
from jax.experimental.pallas import tpu_sc as plsc

K = 1024
ED = 256
B = 1024
T = 6
BI = 128
NB = B // BI

_COMMIT = 0.25
_DIVERGE = 0.1
_BIGF = 3.0e38
_BIGI = 2 ** 30


def _exact_tree(sq):
    L = sq.shape[1]
    rs = sq.reshape(2, 16, 8, L)
    acc = rs[:, 0]
    for j in range(1, 16):
        acc = acc + rs[:, j]
    v = ((acc[:, 0] + acc[:, 4]) + (acc[:, 2] + acc[:, 6])) + (
        (acc[:, 1] + acc[:, 5]) + (acc[:, 3] + acc[:, 7]))
    return v[0:1] + v[1:2]


def _coarse_body(y_ref, x_ref, w_ref, candy_ref):
    xt = x_ref[...].T
    w = w_ref[...]
    g = lax.dot_general(w, xt, (((1,), (0,)), ((), ())),
                        precision=lax.Precision.HIGHEST,
                        preferred_element_type=jnp.float32)
    wn = jnp.sum(w * w, axis=1, keepdims=True)
    d = wn - (g + g)
    nc = K // 128
    dc = d.reshape(nc, 128, BI)
    kkc = lax.broadcasted_iota(jnp.int32, (nc, 128, BI), 1)
    cbase = lax.broadcasted_iota(jnp.int32, (nc, 1, BI), 0) * 128
    vals, gidx = [], []
    for t in range(3):
        m8 = jnp.min(dc, axis=1, keepdims=True)
        i8 = jnp.min(jnp.where(dc == m8, kkc, _BIGI), axis=1,
                     keepdims=True)
        vals.append(m8.reshape(nc, BI))
        gidx.append((i8 + cbase).reshape(nc, BI))
        if t < 2:
            dc = jnp.where(kkc == i8, _BIGF, dc)
    va = jnp.concatenate(vals, axis=0)
    ga = jnp.concatenate(gidx, axis=0)
    for t in range(T):
        m = jnp.min(va, axis=0, keepdims=True)
        it = jnp.min(jnp.where(va == m, ga, _BIGI), axis=0,
                     keepdims=True)
        candy_ref[t:t + 1, :] = it
        if t < T - 1:
            va = jnp.where(ga == it, _BIGF, va)
    candy_ref[T:T + 1, :] = y_ref[0]


def _refine_body(y_ref, yc_ref, x_ref, wc_ref, candy_ref,
                 close_ref, qst_ref, loss_ref, perp_ref,
                 hist_ref, acc_ref):
    ib = pl.program_id(0)

    @pl.when(ib == 0)
    def _init():
        hist_ref[...] = jnp.zeros((1, K), jnp.float32)
        acc_ref[0] = 0.0
        acc_ref[1] = 0.0

    xv = x_ref[...]
    xt = xv.T
    dsq_rows = []
    for t in range(T):
        wct = wc_ref[t].T
        diff = xt - wct
        dsq_rows.append(_exact_tree(diff * diff))
    dsq = jnp.concatenate(dsq_rows, axis=0)
    sd = dsq * lax.rsqrt(dsq)
    cidx = candy_ref[0:T, :]
    m = jnp.min(sd, axis=0, keepdims=True)
    idx = jnp.min(jnp.where(sd == m, cidx, _BIGI), axis=0,
                  keepdims=True)
    dmin = jnp.min(dsq, axis=0, keepdims=True)
    yi = y_ref[0]
    ind = (idx != yi).astype(jnp.float32)
    close_ref[0] = idx
    q = wc_ref[T] - xv
    qst_ref[...] = xv + q
    acc_ref[0] = acc_ref[0] + jnp.sum(q * q)
    acc_ref[1] = acc_ref[1] + jnp.sum(ind * dmin)
    kk2 = lax.broadcasted_iota(jnp.int32, (1, K), 1)
    eq = (yc_ref[...] == kk2).astype(jnp.float32)
    hist_ref[...] = hist_ref[...] + jnp.sum(eq, axis=0, keepdims=True)

    @pl.when(ib == NB - 1)
    def _fin():
        scale = 1.0 / (B * ED)
        loss_ref[0:1, 0:1] = ((1.0 + _COMMIT) * acc_ref[0] * scale
                              - (1.0 + _DIVERGE) * acc_ref[1] * scale
                              ).reshape(1, 1)
        probs = hist_ref[...] * (1.0 / B)
        ent = jnp.sum(probs * jnp.log(probs + 1e-10))
        perp_ref[0:1, 0:1] = jnp.exp(-ent).reshape(1, 1)


_SC_NC = 2
_SC_NS = 16
_NW = _SC_NC * _SC_NS


_NCH = 4


@functools.cache
def _gather_rows_kernel(nrows):
    bpw = nrows // _NW
    cpw = bpw // _NCH
    mesh = plsc.VectorSubcoreMesh(core_axis_name="c", subcore_axis_name="s")

    @functools.partial(
        pl.kernel, mesh=mesh,
        out_type=jax.ShapeDtypeStruct((nrows, ED), jnp.float32),
        scratch_types=[
            pltpu.VMEM((bpw,), jnp.int32),
            pltpu.VMEM((_NCH, cpw, ED), jnp.float32),
        ] + [pltpu.SemaphoreType.DMA] * _NCH,
    )
    def _gather_rows(table_hbm, idx_hbm, out_hbm, idx_v, rows_v, *sems):
        wid = lax.axis_index("s") * _SC_NC + lax.axis_index("c")
        base = wid * bpw
        pltpu.sync_copy(idx_hbm.at[pl.ds(base, bpw)], idx_v)
        cps = [pltpu.async_copy(table_hbm.at[idx_v.at[pl.ds(c * cpw, cpw)]],
                                rows_v.at[c], sems[c])
               for c in range(_NCH)]
        for c in range(_NCH):
            cps[c].wait()
            pltpu.sync_copy(rows_v.at[c], out_hbm.at[pl.ds(base + c * cpw, cpw)])

    return _gather_rows


def kernel(x, y, W):
    y32 = y.astype(jnp.int32)
    y3 = y32.reshape(NB, 1, BI)

    candy = pl.pallas_call(
        _coarse_body,
        grid=(NB,),
        in_specs=[
            pl.BlockSpec((1, 1, BI), lambda ib: (ib, 0, 0)),
            pl.BlockSpec((BI, ED), lambda ib: (ib, 0)),
            pl.BlockSpec((K, ED), lambda ib: (0, 0)),
        ],
        out_specs=pl.BlockSpec((T + 1, BI), lambda ib: (0, ib)),
        out_shape=jax.ShapeDtypeStruct((T + 1, B), jnp.int32),
    )(y3, x, W)

    wc = W[candy.reshape((T + 1) * B)]
    wc = wc.reshape(T + 1, B, ED)

    close3, qst, loss, perp = pl.pallas_call(
        _refine_body,
        grid=(NB,),
        in_specs=[
            pl.BlockSpec((1, 1, BI), lambda ib: (ib, 0, 0)),
            pl.BlockSpec((BI, 1), lambda ib: (ib, 0)),
            pl.BlockSpec((BI, ED), lambda ib: (ib, 0)),
            pl.BlockSpec((T + 1, BI, ED), lambda ib: (0, ib, 0)),
            pl.BlockSpec((T + 1, BI), lambda ib: (0, ib)),
        ],
        out_specs=[
            pl.BlockSpec((1, 1, BI), lambda ib: (ib, 0, 0)),
            pl.BlockSpec((BI, ED), lambda ib: (ib, 0)),
            pl.BlockSpec((1, 1), lambda ib: (0, 0)),
            pl.BlockSpec((1, 1), lambda ib: (0, 0)),
        ],
        out_shape=[
            jax.ShapeDtypeStruct((NB, 1, BI), jnp.int32),
            jax.ShapeDtypeStruct((B, ED), jnp.float32),
            jax.ShapeDtypeStruct((1, 1), jnp.float32),
            jax.ShapeDtypeStruct((1, 1), jnp.float32),
        ],
        scratch_shapes=[
            pltpu.VMEM((1, K), jnp.float32),
            pltpu.SMEM((2,), jnp.float32),
        ],
    )(y3, y32.reshape(B, 1), x, wc, candy)

    return (loss.reshape(()), qst, perp.reshape(()), close3.reshape(B, 1))


# speedup vs baseline: 1.2430x; 1.2430x over previous
import functools

import jax
import jax.numpy as jnp
from jax import lax
from jax.experimental import pallas as pl
from jax.experimental.pallas import tpu as pltpu
from jax.experimental.pallas import tpu_sc as plsc

K = 1024
ED = 256
B = 1024
T = 6
BI = 256
NB = B // BI

_COMMIT = 0.25
_DIVERGE = 0.1
_BIGF = 3.0e38
_BIGI = 2 ** 30


def _exact_tree(sq):
    L = sq.shape[1]
    rs = sq.reshape(2, 16, 8, L)
    acc = rs[:, 0]
    for j in range(1, 16):
        acc = acc + rs[:, j]
    v = ((acc[:, 0] + acc[:, 4]) + (acc[:, 2] + acc[:, 6])) + (
        (acc[:, 1] + acc[:, 5]) + (acc[:, 3] + acc[:, 7]))
    return v[0:1] + v[1:2]


def _coarse_body(y_ref, x_ref, w_ref, candy_ref, wn_ref):
    ib = pl.program_id(0)

    @pl.when(ib == 0)
    def _wn_once():
        w0 = w_ref[...]
        wn_ref[...] = jnp.sum(w0 * w0, axis=1, keepdims=True)

    xt = x_ref[...].T
    w = w_ref[...]
    g = lax.dot_general(w, xt, (((1,), (0,)), ((), ())),
                        precision=lax.Precision.HIGHEST,
                        preferred_element_type=jnp.float32)
    d = wn_ref[...] - (g + g)
    nc = K // 128
    dc = d.reshape(nc, 128, BI)
    kkc = lax.broadcasted_iota(jnp.int32, (nc, 128, BI), 1)
    cbase = lax.broadcasted_iota(jnp.int32, (nc, 1, BI), 0) * 128
    vals, gidx = [], []
    for t in range(3):
        m8 = jnp.min(dc, axis=1, keepdims=True)
        i8 = jnp.min(jnp.where(dc == m8, kkc, _BIGI), axis=1,
                     keepdims=True)
        vals.append(m8.reshape(nc, BI))
        gidx.append((i8 + cbase).reshape(nc, BI))
        if t < 2:
            dc = jnp.where(kkc == i8, _BIGF, dc)
    va = jnp.concatenate(vals, axis=0)
    ga = jnp.concatenate(gidx, axis=0)
    for t in range(T):
        m = jnp.min(va, axis=0, keepdims=True)
        it = jnp.min(jnp.where(va == m, ga, _BIGI), axis=0,
                     keepdims=True)
        candy_ref[t:t + 1, :] = it
        if t < T - 1:
            va = jnp.where(ga == it, _BIGF, va)
    candy_ref[T:T + 1, :] = y_ref[0]


def _refine_body(y_ref, yc_ref, x_ref, wc_ref, candy_ref,
                 close_ref, qst_ref, loss_ref, perp_ref,
                 hist_ref, acc_ref):
    ib = pl.program_id(0)

    @pl.when(ib == 0)
    def _init():
        hist_ref[...] = jnp.zeros((1, K), jnp.float32)
        acc_ref[0] = 0.0
        acc_ref[1] = 0.0

    xv = x_ref[...]
    xt = xv.T
    dsq_rows = []
    for t in range(T):
        wct = wc_ref[t].T
        diff = xt - wct
        dsq_rows.append(_exact_tree(diff * diff))
    dsq = jnp.concatenate(dsq_rows, axis=0)
    sd = dsq * lax.rsqrt(dsq)
    cidx = candy_ref[0:T, :]
    m = jnp.min(sd, axis=0, keepdims=True)
    idx = jnp.min(jnp.where(sd == m, cidx, _BIGI), axis=0,
                  keepdims=True)
    dmin = jnp.min(dsq, axis=0, keepdims=True)
    yi = y_ref[0]
    ind = (idx != yi).astype(jnp.float32)
    close_ref[0] = idx
    q = wc_ref[T] - xv
    qst_ref[...] = xv + q
    acc_ref[0] = acc_ref[0] + jnp.sum(q * q)
    acc_ref[1] = acc_ref[1] + jnp.sum(ind * dmin)
    kk2 = lax.broadcasted_iota(jnp.int32, (1, K), 1)
    eq = (yc_ref[...] == kk2).astype(jnp.float32)
    hist_ref[...] = hist_ref[...] + jnp.sum(eq, axis=0, keepdims=True)

    @pl.when(ib == NB - 1)
    def _fin():
        scale = 1.0 / (B * ED)
        loss_ref[0:1, 0:1] = ((1.0 + _COMMIT) * acc_ref[0] * scale
                              - (1.0 + _DIVERGE) * acc_ref[1] * scale
                              ).reshape(1, 1)
        probs = hist_ref[...] * (1.0 / B)
        ent = jnp.sum(probs * jnp.log(probs + 1e-10))
        perp_ref[0:1, 0:1] = jnp.exp(-ent).reshape(1, 1)


_SC_NC = 2
_SC_NS = 16
_NW = _SC_NC * _SC_NS


_NCH = 4


@functools.cache
def _gather_rows_kernel(nrows):
    bpw = nrows // _NW
    cpw = bpw // _NCH
    mesh = plsc.VectorSubcoreMesh(core_axis_name="c", subcore_axis_name="s")

    @functools.partial(
        pl.kernel, mesh=mesh,
        out_type=jax.ShapeDtypeStruct((nrows, ED), jnp.float32),
        scratch_types=[
            pltpu.VMEM((bpw,), jnp.int32),
            pltpu.VMEM((_NCH, cpw, ED), jnp.float32),
        ] + [pltpu.SemaphoreType.DMA] * _NCH,
    )
    def _gather_rows(table_hbm, idx_hbm, out_hbm, idx_v, rows_v, *sems):
        wid = lax.axis_index("s") * _SC_NC + lax.axis_index("c")
        base = wid * bpw
        pltpu.sync_copy(idx_hbm.at[pl.ds(base, bpw)], idx_v)
        cps = [pltpu.async_copy(table_hbm.at[idx_v.at[pl.ds(c * cpw, cpw)]],
                                rows_v.at[c], sems[c])
               for c in range(_NCH)]
        for c in range(_NCH):
            cps[c].wait()
            pltpu.sync_copy(rows_v.at[c], out_hbm.at[pl.ds(base + c * cpw, cpw)])

    return _gather_rows


def kernel(x, y, W):
    y32 = y.astype(jnp.int32)
    y3 = y32.reshape(NB, 1, BI)

    candy = pl.pallas_call(
        _coarse_body,
        grid=(NB,),
        in_specs=[
            pl.BlockSpec((1, 1, BI), lambda ib: (ib, 0, 0)),
            pl.BlockSpec((BI, ED), lambda ib: (ib, 0)),
            pl.BlockSpec((K, ED), lambda ib: (0, 0)),
        ],
        out_specs=pl.BlockSpec((T + 1, BI), lambda ib: (0, ib)),
        out_shape=jax.ShapeDtypeStruct((T + 1, B), jnp.int32),
        scratch_shapes=[pltpu.VMEM((K, 1), jnp.float32)],
    )(y3, x, W)

    wc = _gather_rows_kernel((T + 1) * B)(W, candy.reshape((T + 1) * B))
    wc = wc.reshape(T + 1, B, ED)

    close3, qst, loss, perp = pl.pallas_call(
        _refine_body,
        grid=(NB,),
        in_specs=[
            pl.BlockSpec((1, 1, BI), lambda ib: (ib, 0, 0)),
            pl.BlockSpec((BI, 1), lambda ib: (ib, 0)),
            pl.BlockSpec((BI, ED), lambda ib: (ib, 0)),
            pl.BlockSpec((T + 1, BI, ED), lambda ib: (0, ib, 0)),
            pl.BlockSpec((T + 1, BI), lambda ib: (0, ib)),
        ],
        out_specs=[
            pl.BlockSpec((1, 1, BI), lambda ib: (ib, 0, 0)),
            pl.BlockSpec((BI, ED), lambda ib: (ib, 0)),
            pl.BlockSpec((1, 1), lambda ib: (0, 0)),
            pl.BlockSpec((1, 1), lambda ib: (0, 0)),
        ],
        out_shape=[
            jax.ShapeDtypeStruct((NB, 1, BI), jnp.int32),
            jax.ShapeDtypeStruct((B, ED), jnp.float32),
            jax.ShapeDtypeStruct((1, 1), jnp.float32),
            jax.ShapeDtypeStruct((1, 1), jnp.float32),
        ],
        scratch_shapes=[
            pltpu.VMEM((1, K), jnp.float32),
            pltpu.SMEM((2,), jnp.float32),
        ],
    )(y3, y32.reshape(B, 1), x, wc, candy)

    return (loss.reshape(()), qst, perp.reshape(()), close3.reshape(B, 1))
